# baseline (speedup 1.0000x reference)
import functools

import jax
import jax.numpy as jnp
from jax import lax
from jax.experimental import pallas as pl
from jax.experimental.pallas import tpu as pltpu
from jax.experimental.pallas import tpu_sc as plsc

B = 1024 * 200
DW = 128
DE = 64
NW = 32
PER_W = B // NW
C = 640
NITER = PER_W // C
GSUB = C // 128

_mesh = plsc.VectorSubcoreMesh(core_axis_name="c", subcore_axis_name="s")


@functools.partial(
    pl.kernel,
    out_type=jax.ShapeDtypeStruct((B, DW + DE), jnp.float32),
    mesh=_mesh,
    scratch_types=[
        pltpu.VMEM((C,), jnp.int32),
        pltpu.VMEM((C, DE), jnp.float32),
        pltpu.SemaphoreType.DMA,
        pltpu.SemaphoreType.DMA,
    ],
    compiler_params=pltpu.CompilerParams(use_tc_tiling_on_sc=False),
)
def _concat_gather(wall_hbm, idx_hbm, table_hbm, out_hbm, idx_v, rows_v, sem, wsem):
    wid = lax.axis_index("s") * 2 + lax.axis_index("c")
    base = wid * PER_W

    wall_cp = pltpu.async_copy(
        wall_hbm.at[pl.ds(base, PER_W), :],
        out_hbm.at[pl.ds(base, PER_W), pl.ds(0, DW)],
        wsem,
    )

    def body(i, carry):
        off = base + i * C
        pltpu.sync_copy(idx_hbm.at[pl.ds(off, C)], idx_v)
        gathers = [
            pltpu.async_copy(
                table_hbm.at[idx_v.at[pl.ds(j * 128, 128)]],
                rows_v.at[pl.ds(j * 128, 128), :],
                sem,
            )
            for j in range(GSUB)
        ]
        for g in gathers:
            g.wait()
        pltpu.sync_copy(rows_v, out_hbm.at[pl.ds(off, C), pl.ds(DW, DE)])
        return carry

    lax.fori_loop(0, NITER, body, 0)
    wall_cp.wait()


def kernel(wall_distances, keymask, key_embed):
    wall2d = wall_distances.reshape(B, DW)
    idx1d = keymask.reshape(B)
    out = _concat_gather(wall2d, idx1d, key_embed)
    return out.reshape(1024, 200, DW + DE)


# speedup vs baseline: 6.9707x; 6.9707x over previous
import functools

import jax
import jax.numpy as jnp
from jax import lax
from jax.experimental import pallas as pl
from jax.experimental.pallas import tpu as pltpu
from jax.experimental.pallas import tpu_sc as plsc

B = 1024 * 200
DW = 128
DE = 64
NW = 32
PER_W = B // NW
C = 128
NITER = PER_W // C
NBUF = 5
LA = 2

_mesh = plsc.VectorSubcoreMesh(core_axis_name="c", subcore_axis_name="s")


@functools.partial(
    pl.kernel,
    out_type=jax.ShapeDtypeStruct((B, DW + DE), jnp.float32),
    mesh=_mesh,
    scratch_types=[
        pltpu.VMEM((PER_W,), jnp.int32),
        pltpu.VMEM((NBUF, C, DW), jnp.float32),
        pltpu.VMEM((NBUF, C, DE), jnp.float32),
        pltpu.SemaphoreType.DMA((NBUF,)),
        pltpu.SemaphoreType.DMA((NBUF,)),
    ],
    compiler_params=pltpu.CompilerParams(use_tc_tiling_on_sc=False),
)
def _concat_gather(wall_hbm, idx_hbm, table_hbm, out_hbm, idx_v, wall_v, rows_v,
                   lsem, ssem):
    wid = lax.axis_index("s") * 2 + lax.axis_index("c")
    base = wid * PER_W

    def fire_load(ci, b):
        pltpu.async_copy(
            wall_hbm.at[pl.ds(base + ci * C, C), :], wall_v.at[b], lsem.at[b])

    def wait_load(b):
        pltpu.make_async_copy(
            wall_hbm.at[pl.ds(0, C), :], wall_v.at[b], lsem.at[b]).wait()

    def fire_stores(ci, b):
        off = base + ci * C
        pltpu.async_copy(
            wall_v.at[b], out_hbm.at[pl.ds(off, C), pl.ds(0, DW)], ssem.at[b])
        pltpu.async_copy(
            rows_v.at[b], out_hbm.at[pl.ds(off, C), pl.ds(DW, DE)], ssem.at[b])

    def wait_stores(b):
        pltpu.make_async_copy(
            wall_v.at[b], out_hbm.at[pl.ds(0, C), pl.ds(0, DW)], ssem.at[b]).wait()
        pltpu.make_async_copy(
            rows_v.at[b], out_hbm.at[pl.ds(0, C), pl.ds(DW, DE)], ssem.at[b]).wait()

    def step(ci, b, wait_prev_store, fire_next_load):
        wait_load(b)
        gather = pltpu.async_copy(
            table_hbm.at[idx_v.at[pl.ds(ci * C, C)]], rows_v.at[b], lsem.at[b])
        if fire_next_load:
            nb = (b + LA) % NBUF
            if wait_prev_store:
                wait_stores(nb)
            fire_load(ci + LA, nb)
        gather.wait()
        fire_stores(ci, b)

    pltpu.sync_copy(idx_hbm.at[pl.ds(base, PER_W)], idx_v)

    for ci in range(LA):
        fire_load(ci, ci)

    for ci in range(NBUF):
        step(ci, ci % NBUF, wait_prev_store=(ci + LA >= NBUF),
             fire_next_load=True)

    def body(k, carry):
        for b in range(NBUF):
            step(k * NBUF + b, b, wait_prev_store=True, fire_next_load=True)
        return carry

    lax.fori_loop(1, NITER // NBUF - 1, body, 0)

    for ci in range(NITER - NBUF, NITER):
        step(ci, ci % NBUF, wait_prev_store=True,
             fire_next_load=(ci + LA < NITER))

    for b in range(NBUF):
        wait_stores(b)


def kernel(wall_distances, keymask, key_embed):
    wall2d = wall_distances.reshape(B, DW)
    idx1d = keymask.reshape(B)
    out = _concat_gather(wall2d, idx1d, key_embed)
    return out.reshape(1024, 200, DW + DE)


# speedup vs baseline: 8.4093x; 1.2064x over previous
import functools

import jax
import jax.numpy as jnp
from jax import lax
from jax.experimental import pallas as pl
from jax.experimental.pallas import tpu as pltpu
from jax.experimental.pallas import tpu_sc as plsc

B = 1024 * 200
DW = 128
DE = 64
DP = 128
NW = 32
PER_W = B // NW
C = 128
NITER = PER_W // C
NBUF = 5
LA = 2

_mesh = plsc.VectorSubcoreMesh(core_axis_name="c", subcore_axis_name="s")


@functools.partial(
    pl.kernel,
    out_type=jax.ShapeDtypeStruct((B, DP), jnp.float32),
    mesh=_mesh,
    scratch_types=[
        pltpu.VMEM((PER_W,), jnp.int32),
        pltpu.VMEM((NBUF, C, DP), jnp.float32),
        pltpu.SemaphoreType.DMA((NBUF,)),
        pltpu.SemaphoreType.DMA((NBUF,)),
    ],
    compiler_params=pltpu.CompilerParams(use_tc_tiling_on_sc=True),
)
def _sc_gather(idx_hbm, table_hbm, gath_hbm, idx_v, rows_v, gsem, ssem):
    wid = lax.axis_index("s") * 2 + lax.axis_index("c")
    base = wid * PER_W

    def fire_gather(ci, b):
        pltpu.async_copy(
            table_hbm.at[idx_v.at[pl.ds(ci * C, C)]], rows_v.at[b], gsem.at[b])

    def wait_gather(b):
        pltpu.make_async_copy(
            table_hbm.at[idx_v.at[pl.ds(0, C)]], rows_v.at[b], gsem.at[b]).wait()

    def fire_store(ci, b):
        pltpu.async_copy(
            rows_v.at[b], gath_hbm.at[pl.ds(base + ci * C, C), :], ssem.at[b])

    def wait_store(b):
        pltpu.make_async_copy(
            rows_v.at[b], gath_hbm.at[pl.ds(0, C), :], ssem.at[b]).wait()

    def step(ci, b, wait_prev_store, fire_next_gather):
        wait_gather(b)
        if fire_next_gather:
            nb = (b + LA) % NBUF
            if wait_prev_store:
                wait_store(nb)
            fire_gather(ci + LA, nb)
        fire_store(ci, b)

    pltpu.sync_copy(idx_hbm.at[pl.ds(base, PER_W)], idx_v)

    for ci in range(LA):
        fire_gather(ci, ci)

    for ci in range(NBUF):
        step(ci, ci % NBUF, wait_prev_store=(ci + LA >= NBUF),
             fire_next_gather=True)

    def body(k, carry):
        for b in range(NBUF):
            step(k * NBUF + b, b, wait_prev_store=True, fire_next_gather=True)
        return carry

    lax.fori_loop(1, NITER // NBUF - 1, body, 0)

    for ci in range(NITER - NBUF, NITER):
        step(ci, ci % NBUF, wait_prev_store=True,
             fire_next_gather=(ci + LA < NITER))

    for b in range(NBUF):
        wait_store(b)


RB = 2048


def _tc_concat_body(wall_ref, gath_ref, out_ref):
    out_ref[:, 0:DW] = wall_ref[...]
    out_ref[:, DW:DW + DE] = gath_ref[:, 0:DE]


_tc_concat = pl.pallas_call(
    _tc_concat_body,
    out_shape=jax.ShapeDtypeStruct((B, DW + DE), jnp.float32),
    grid=(B // RB,),
    in_specs=[
        pl.BlockSpec((RB, DW), lambda i: (i, 0)),
        pl.BlockSpec((RB, DP), lambda i: (i, 0)),
    ],
    out_specs=pl.BlockSpec((RB, DW + DE), lambda i: (i, 0)),
)


def kernel(wall_distances, keymask, key_embed):
    wall2d = wall_distances.reshape(B, DW)
    idx1d = keymask.reshape(B)
    table_pad = jnp.pad(key_embed, ((0, 0), (0, DP - DE)))
    gath = _sc_gather(idx1d, table_pad)
    out = _tc_concat(wall2d, gath)
    return out.reshape(1024, 200, DW + DE)
